# baseline (speedup 1.0000x reference)
import functools
import jax
import jax.numpy as jnp
from jax import lax
from jax.experimental import pallas as pl
from jax.experimental.pallas import tpu as pltpu
from jax.experimental.pallas import tpu_sc as plsc

D = 2048
NTOK = 4 * 4096
T = 16
ROWS_PER_W = NTOK // 32
NGROUPS = ROWS_PER_W // T


def _sc_body(w_hbm, x_hbm, o_hbm, w_v, x_v, o_v, p2_v, p4_v, p8_v):
    nc = 2
    wid = lax.axis_index("s") * nc + lax.axis_index("c")
    pltpu.sync_copy(w_hbm, w_v)
    u2 = w_v[pl.ds(0, 16)]
    u4 = w_v[pl.ds(16, 16)]
    u8 = w_v[pl.ds(32, 16)]
    d2 = w_v[pl.ds(48, 16)]
    d4 = w_v[pl.ds(64, 16)]
    d8 = w_v[pl.ds(80, 16)]
    col = lax.iota(jnp.int32, 16) * D

    def jblock(jb, downs, skip_first=False):
        j0 = jb * 8
        p2 = p4 = p8 = None
        for t in (7, 6, 5, 4, 3, 2, 1, 0):
            j = j0 + t
            v = plsc.load_gather(x_v, [col + j])
            if t % 2 == 1:
                p2 = v
            else:
                p2 = p2 + v
                p2_v[pl.ds((j >> 1) * 16, 16)] = p2
                if t in (6, 2):
                    p4 = p2
                else:
                    p4 = p4 + p2
                    p4_v[pl.ds((j >> 2) * 16, 16)] = p4
                    if t == 4:
                        p8 = p4
                    else:
                        p8 = p8 + p4
                        p8_v[pl.ds((j >> 3) * 16, 16)] = p8
            acc = v
            if t % 2 == 0:
                acc = acc + u2 * plsc.load_gather(x_v, [col + (j >> 1)])
            if t % 4 == 0:
                acc = acc + u4 * plsc.load_gather(x_v, [col + (j >> 2)])
            if t % 8 == 0:
                acc = acc + u8 * plsc.load_gather(x_v, [col + (j >> 3)])
            if not (skip_first and t == 0):
                if 2 in downs:
                    acc = acc + d2 * p2_v[pl.ds(j * 16, 16)]
                if 4 in downs:
                    acc = acc + d4 * p4_v[pl.ds(j * 16, 16)]
                if 8 in downs:
                    acc = acc + d8 * p8_v[pl.ds(j * 16, 16)]
            plsc.store_scatter(o_v, [col + j], acc)

    def do_group(g, _):
        row0 = wid * ROWS_PER_W + g * T
        pltpu.sync_copy(x_hbm.at[pl.ds(row0 * D, T * D)], x_v)
        lax.fori_loop(0, 128, lambda i, _: (jblock(255 - i, ()), 0)[1], 0)
        lax.fori_loop(0, 64, lambda i, _: (jblock(127 - i, (2,)), 0)[1], 0)
        lax.fori_loop(0, 32, lambda i, _: (jblock(63 - i, (2, 4)), 0)[1], 0)
        lax.fori_loop(0, 31, lambda i, _: (jblock(31 - i, (2, 4, 8)), 0)[1], 0)
        jblock(0, (2, 4, 8), skip_first=True)
        pltpu.sync_copy(o_v, o_hbm.at[pl.ds(row0 * D, T * D)])
        return 0

    lax.fori_loop(0, NGROUPS, do_group, 0)


def kernel(x, up_weights, down_weights):
    B, S, d = x.shape
    xf = x.reshape(B * S * d)
    w = jnp.concatenate([jax.nn.sigmoid(up_weights), jax.nn.sigmoid(down_weights)])
    wb = jnp.broadcast_to(w[:, None], (6, 16)).reshape(96)
    mesh = plsc.VectorSubcoreMesh(core_axis_name="c", subcore_axis_name="s")
    run = functools.partial(
        pl.kernel,
        mesh=mesh,
        compiler_params=pltpu.CompilerParams(needs_layout_passes=False),
        out_type=jax.ShapeDtypeStruct((B * S * d,), jnp.float32),
        scratch_types=[
            pltpu.VMEM((96,), jnp.float32),
            pltpu.VMEM((T * D,), jnp.float32),
            pltpu.VMEM((T * D,), jnp.float32),
            pltpu.VMEM((1024 * 16,), jnp.float32),
            pltpu.VMEM((512 * 16,), jnp.float32),
            pltpu.VMEM((256 * 16,), jnp.float32),
        ],
    )(_sc_body)
    out = run(wb, xf)
    return out.reshape(B, S, d)


# speedup vs baseline: 1.5699x; 1.5699x over previous
import functools
import jax
import jax.numpy as jnp
from jax import lax
from jax.experimental import pallas as pl
from jax.experimental.pallas import tpu as pltpu
from jax.experimental.pallas import tpu_sc as plsc

D = 2048
P = 2049
NTOK = 4 * 4096
T = 16
ROWS_PER_W = NTOK // 32
NGROUPS = ROWS_PER_W // T


def _sc_body(w_hbm, x_hbm, o_hbm, w_v, x_v, x_p, o_p, p2_v, p4_v, p8_v):
    nc = 2
    wid = lax.axis_index("s") * nc + lax.axis_index("c")
    pltpu.sync_copy(w_hbm, w_v)
    u2 = w_v[pl.ds(0, 16)]
    u4 = w_v[pl.ds(16, 16)]
    u8 = w_v[pl.ds(32, 16)]
    d2 = w_v[pl.ds(48, 16)]
    d4 = w_v[pl.ds(64, 16)]
    d8 = w_v[pl.ds(80, 16)]
    lane = lax.iota(jnp.int32, 16)
    col = lane * P

    def jblock(jb, downs, skip_first=False):
        j0 = jb * 8
        p2 = p4 = p8 = None
        for t in (7, 6, 5, 4, 3, 2, 1, 0):
            j = j0 + t
            v = plsc.load_gather(x_p, [col + j])
            if t % 2 == 1:
                p2 = v
            else:
                p2 = p2 + v
                p2_v[pl.ds((j >> 1) * 16, 16)] = p2
                if t in (6, 2):
                    p4 = p2
                else:
                    p4 = p4 + p2
                    p4_v[pl.ds((j >> 2) * 16, 16)] = p4
                    if t == 4:
                        p8 = p4
                    else:
                        p8 = p8 + p4
                        p8_v[pl.ds((j >> 3) * 16, 16)] = p8
            acc = v
            if t % 2 == 0:
                acc = acc + u2 * plsc.load_gather(x_p, [col + (j >> 1)])
            if t % 4 == 0:
                acc = acc + u4 * plsc.load_gather(x_p, [col + (j >> 2)])
            if t % 8 == 0:
                acc = acc + u8 * plsc.load_gather(x_p, [col + (j >> 3)])
            if not (skip_first and t == 0):
                if 2 in downs:
                    acc = acc + d2 * p2_v[pl.ds(j * 16, 16)]
                if 4 in downs:
                    acc = acc + d4 * p4_v[pl.ds(j * 16, 16)]
                if 8 in downs:
                    acc = acc + d8 * p8_v[pl.ds(j * 16, 16)]
            plsc.store_scatter(o_p, [col + j], acc)

    def respread(c, _):
        for r in range(T):
            v = x_v[pl.ds(r * D + c * 16, 16)]
            plsc.store_scatter(x_p, [r * P + c * 16 + lane], v)
        return 0

    def unspread(c, _):
        for r in range(T):
            v = plsc.load_gather(o_p, [r * P + c * 16 + lane])
            x_v[pl.ds(r * D + c * 16, 16)] = v
        return 0

    def do_group(g, _):
        row0 = wid * ROWS_PER_W + g * T
        pltpu.sync_copy(x_hbm.at[pl.ds(row0 * D, T * D)], x_v)
        lax.fori_loop(0, D // 16, respread, 0)
        lax.fori_loop(0, 128, lambda i, _: (jblock(255 - i, ()), 0)[1], 0)
        lax.fori_loop(0, 64, lambda i, _: (jblock(127 - i, (2,)), 0)[1], 0)
        lax.fori_loop(0, 32, lambda i, _: (jblock(63 - i, (2, 4)), 0)[1], 0)
        lax.fori_loop(0, 31, lambda i, _: (jblock(31 - i, (2, 4, 8)), 0)[1], 0)
        jblock(0, (2, 4, 8), skip_first=True)
        lax.fori_loop(0, D // 16, unspread, 0)
        pltpu.sync_copy(x_v, o_hbm.at[pl.ds(row0 * D, T * D)])
        return 0

    lax.fori_loop(0, NGROUPS, do_group, 0)


def kernel(x, up_weights, down_weights):
    B, S, d = x.shape
    xf = x.reshape(B * S * d)
    w = jnp.concatenate([jax.nn.sigmoid(up_weights), jax.nn.sigmoid(down_weights)])
    wb = jnp.broadcast_to(w[:, None], (6, 16)).reshape(96)
    mesh = plsc.VectorSubcoreMesh(core_axis_name="c", subcore_axis_name="s")
    run = functools.partial(
        pl.kernel,
        mesh=mesh,
        compiler_params=pltpu.CompilerParams(needs_layout_passes=False),
        out_type=jax.ShapeDtypeStruct((B * S * d,), jnp.float32),
        scratch_types=[
            pltpu.VMEM((96,), jnp.float32),
            pltpu.VMEM((T * D,), jnp.float32),
            pltpu.VMEM((T * P,), jnp.float32),
            pltpu.VMEM((T * P,), jnp.float32),
            pltpu.VMEM((1024 * 16,), jnp.float32),
            pltpu.VMEM((512 * 16,), jnp.float32),
            pltpu.VMEM((256 * 16,), jnp.float32),
        ],
    )(_sc_body)
    out = run(wb, xf)
    return out.reshape(B, S, d)


# speedup vs baseline: 3.2944x; 2.0985x over previous
import functools
import jax
import jax.numpy as jnp
from jax import lax
from jax.experimental import pallas as pl
from jax.experimental.pallas import tpu as pltpu
from jax.experimental.pallas import tpu_sc as plsc

D = 2048
P = 2049
NTOK = 4 * 4096
T = 16
ROWS_PER_W = NTOK // 32
NGROUPS = ROWS_PER_W // T


def _sc_body(w_hbm, x_hbm, o_hbm, w_v, x_v, x_p, o_p, p2_v, p4_v, p8_v):
    nc = 2
    wid = lax.axis_index("s") * nc + lax.axis_index("c")
    pltpu.sync_copy(w_hbm, w_v)
    u2 = w_v[pl.ds(0, 16)]
    u4 = w_v[pl.ds(16, 16)]
    u8 = w_v[pl.ds(32, 16)]
    d2 = w_v[pl.ds(48, 16)]
    d4 = w_v[pl.ds(64, 16)]
    d8 = w_v[pl.ds(80, 16)]
    lane = lax.iota(jnp.int32, 16)
    col = lane * P

    def jblock(jb, downs, skip_first=False):
        j0 = jb * 8
        p2 = p4 = p8 = None
        for t in (7, 6, 5, 4, 3, 2, 1, 0):
            j = j0 + t
            v = plsc.load_gather(x_p, [col + j])
            if t % 2 == 1:
                p2 = v
            else:
                p2 = p2 + v
                p2_v[pl.ds((j >> 1) * 16, 16)] = p2
                if t in (6, 2):
                    p4 = p2
                else:
                    p4 = p4 + p2
                    p4_v[pl.ds((j >> 2) * 16, 16)] = p4
                    if t == 4:
                        p8 = p4
                    else:
                        p8 = p8 + p4
                        p8_v[pl.ds((j >> 3) * 16, 16)] = p8
            acc = v
            if t % 2 == 0:
                acc = acc + u2 * plsc.load_gather(x_p, [col + (j >> 1)])
            if t % 4 == 0:
                acc = acc + u4 * plsc.load_gather(x_p, [col + (j >> 2)])
            if t % 8 == 0:
                acc = acc + u8 * plsc.load_gather(x_p, [col + (j >> 3)])
            if not (skip_first and t == 0):
                if 2 in downs:
                    acc = acc + d2 * p2_v[pl.ds(j * 16, 16)]
                if 4 in downs:
                    acc = acc + d4 * p4_v[pl.ds(j * 16, 16)]
                if 8 in downs:
                    acc = acc + d8 * p8_v[pl.ds(j * 16, 16)]
            plsc.store_scatter(o_p, [col + j], acc)

    def do_group(g, _):
        row0 = wid * ROWS_PER_W + g * T
        pltpu.sync_copy(x_hbm.at[pl.ds(row0 * D, T * D)], x_v)

        @plsc.parallel_loop(0, D // 16, unroll=2)
        def respread(c):
            for r in range(T):
                v = x_v[pl.ds(r * D + c * 16, 16)]
                plsc.store_scatter(x_p, [r * P + c * 16 + lane], v)

        for lo, hi, downs in (
            (128, 256, ()),
            (64, 128, (2,)),
            (32, 64, (2, 4)),
            (16, 32, (2, 4, 8)),
            (8, 16, (2, 4, 8)),
            (4, 8, (2, 4, 8)),
            (2, 4, (2, 4, 8)),
        ):
            @plsc.parallel_loop(lo, hi, unroll=2)
            def sweep(jb, downs=downs):
                jblock(jb, downs)

        jblock(1, (2, 4, 8))
        jblock(0, (2, 4, 8), skip_first=True)

        @plsc.parallel_loop(0, D // 16, unroll=2)
        def unspread(c):
            for r in range(T):
                v = plsc.load_gather(o_p, [r * P + c * 16 + lane])
                x_v[pl.ds(r * D + c * 16, 16)] = v

        pltpu.sync_copy(x_v, o_hbm.at[pl.ds(row0 * D, T * D)])
        return 0

    lax.fori_loop(0, NGROUPS, do_group, 0)


def kernel(x, up_weights, down_weights):
    B, S, d = x.shape
    xf = x.reshape(B * S * d)
    w = jnp.concatenate([jax.nn.sigmoid(up_weights), jax.nn.sigmoid(down_weights)])
    wb = jnp.broadcast_to(w[:, None], (6, 16)).reshape(96)
    mesh = plsc.VectorSubcoreMesh(core_axis_name="c", subcore_axis_name="s")
    run = functools.partial(
        pl.kernel,
        mesh=mesh,
        compiler_params=pltpu.CompilerParams(needs_layout_passes=False),
        out_type=jax.ShapeDtypeStruct((B * S * d,), jnp.float32),
        scratch_types=[
            pltpu.VMEM((96,), jnp.float32),
            pltpu.VMEM((T * D,), jnp.float32),
            pltpu.VMEM((T * P,), jnp.float32),
            pltpu.VMEM((T * P,), jnp.float32),
            pltpu.VMEM((1024 * 16,), jnp.float32),
            pltpu.VMEM((512 * 16,), jnp.float32),
            pltpu.VMEM((256 * 16,), jnp.float32),
        ],
    )(_sc_body)
    out = run(wb, xf)
    return out.reshape(B, S, d)


# speedup vs baseline: 3.5428x; 1.0754x over previous
import functools
import jax
import jax.numpy as jnp
from jax import lax
from jax.experimental import pallas as pl
from jax.experimental.pallas import tpu as pltpu
from jax.experimental.pallas import tpu_sc as plsc

D = 2048
P = 2049
NTOK = 4 * 4096
T = 16
ROWS_PER_W = NTOK // 32
NGROUPS = ROWS_PER_W // T


def _sc_body(w_hbm, x_hbm, o_hbm, w_v, x_v, x_p, o_p, p2_v, p4_v, p8_v):
    nc = 2
    wid = lax.axis_index("s") * nc + lax.axis_index("c")
    pltpu.sync_copy(w_hbm, w_v)
    u2 = w_v[pl.ds(0, 16)]
    u4 = w_v[pl.ds(16, 16)]
    u8 = w_v[pl.ds(32, 16)]
    d2 = w_v[pl.ds(48, 16)]
    d4 = w_v[pl.ds(64, 16)]
    d8 = w_v[pl.ds(80, 16)]
    lane = lax.iota(jnp.int32, 16)
    col = lane * P

    def jblock(jb, downs, skip_first=False):
        j0 = jb * 8
        p2 = p4 = p8 = None
        for t in (7, 6, 5, 4, 3, 2, 1, 0):
            j = j0 + t
            v = plsc.load_gather(x_p, [col + j])
            if t % 2 == 1:
                p2 = v
            else:
                p2 = p2 + v
                p2_v[pl.ds((j >> 1) * 16, 16)] = p2
                if t in (6, 2):
                    p4 = p2
                else:
                    p4 = p4 + p2
                    p4_v[pl.ds((j >> 2) * 16, 16)] = p4
                    if t == 4:
                        p8 = p4
                    else:
                        p8 = p8 + p4
                        p8_v[pl.ds((j >> 3) * 16, 16)] = p8
            acc = v
            if t % 2 == 0:
                acc = acc + u2 * plsc.load_gather(x_p, [col + (j >> 1)])
            if t % 4 == 0:
                acc = acc + u4 * plsc.load_gather(x_p, [col + (j >> 2)])
            if t % 8 == 0:
                acc = acc + u8 * plsc.load_gather(x_p, [col + (j >> 3)])
            if not (skip_first and t == 0):
                if 2 in downs:
                    acc = acc + d2 * p2_v[pl.ds(j * 16, 16)]
                if 4 in downs:
                    acc = acc + d4 * p4_v[pl.ds(j * 16, 16)]
                if 8 in downs:
                    acc = acc + d8 * p8_v[pl.ds(j * 16, 16)]
            plsc.store_scatter(o_p, [col + j], acc)

    def do_group(g, _):
        row0 = wid * ROWS_PER_W + g * T
        pltpu.sync_copy(x_hbm.at[pl.ds(row0 * D, T * D)], x_v)

        @plsc.parallel_loop(0, D // 16, unroll=4)
        def respread(c):
            for r in range(T):
                v = x_v[pl.ds(r * D + c * 16, 16)]
                plsc.store_scatter(x_p, [r * P + c * 16 + lane], v)

        for lo, hi, downs in (
            (128, 256, ()),
            (64, 128, (2,)),
            (32, 64, (2, 4)),
            (16, 32, (2, 4, 8)),
            (8, 16, (2, 4, 8)),
            (4, 8, (2, 4, 8)),
            (2, 4, (2, 4, 8)),
        ):
            @plsc.parallel_loop(lo, hi, unroll=4)
            def sweep(jb, downs=downs):
                jblock(jb, downs)

        jblock(1, (2, 4, 8))
        jblock(0, (2, 4, 8), skip_first=True)

        @plsc.parallel_loop(0, D // 16, unroll=4)
        def unspread(c):
            for r in range(T):
                v = plsc.load_gather(o_p, [r * P + c * 16 + lane])
                x_v[pl.ds(r * D + c * 16, 16)] = v

        pltpu.sync_copy(x_v, o_hbm.at[pl.ds(row0 * D, T * D)])
        return 0

    lax.fori_loop(0, NGROUPS, do_group, 0)


def kernel(x, up_weights, down_weights):
    B, S, d = x.shape
    xf = x.reshape(B * S * d)
    w = jnp.concatenate([jax.nn.sigmoid(up_weights), jax.nn.sigmoid(down_weights)])
    wb = jnp.broadcast_to(w[:, None], (6, 16)).reshape(96)
    mesh = plsc.VectorSubcoreMesh(core_axis_name="c", subcore_axis_name="s")
    run = functools.partial(
        pl.kernel,
        mesh=mesh,
        compiler_params=pltpu.CompilerParams(needs_layout_passes=False),
        out_type=jax.ShapeDtypeStruct((B * S * d,), jnp.float32),
        scratch_types=[
            pltpu.VMEM((96,), jnp.float32),
            pltpu.VMEM((T * D,), jnp.float32),
            pltpu.VMEM((T * P,), jnp.float32),
            pltpu.VMEM((T * P,), jnp.float32),
            pltpu.VMEM((1024 * 16,), jnp.float32),
            pltpu.VMEM((512 * 16,), jnp.float32),
            pltpu.VMEM((256 * 16,), jnp.float32),
        ],
    )(_sc_body)
    out = run(wb, xf)
    return out.reshape(B, S, d)


# speedup vs baseline: 4.1673x; 1.1763x over previous
import functools
import jax
import jax.numpy as jnp
from jax import lax
from jax.experimental import pallas as pl
from jax.experimental.pallas import tpu as pltpu
from jax.experimental.pallas import tpu_sc as plsc

D = 2048
P = 2049
T = 16
R = 256
STRIDES = (2, 4, 8)
SC_ROWS = 6656


def _sc_body(rows_per_w, w_hbm, x_hbm, o_hbm, w_v, x_v, x_p, o_p, p2_v, p4_v, p8_v):
    nc = 2
    wid = lax.axis_index("s") * nc + lax.axis_index("c")
    pltpu.sync_copy(w_hbm, w_v)
    u2 = w_v[pl.ds(0, 16)]
    u4 = w_v[pl.ds(16, 16)]
    u8 = w_v[pl.ds(32, 16)]
    d2 = w_v[pl.ds(48, 16)]
    d4 = w_v[pl.ds(64, 16)]
    d8 = w_v[pl.ds(80, 16)]
    lane = lax.iota(jnp.int32, 16)
    col = lane * P

    def jblock(jb, downs, skip_first=False):
        j0 = jb * 8
        p2 = p4 = p8 = None
        for t in (7, 6, 5, 4, 3, 2, 1, 0):
            j = j0 + t
            v = plsc.load_gather(x_p, [col + j])
            if t % 2 == 1:
                p2 = v
            else:
                p2 = p2 + v
                p2_v[pl.ds((j >> 1) * 16, 16)] = p2
                if t in (6, 2):
                    p4 = p2
                else:
                    p4 = p4 + p2
                    p4_v[pl.ds((j >> 2) * 16, 16)] = p4
                    if t == 4:
                        p8 = p4
                    else:
                        p8 = p8 + p4
                        p8_v[pl.ds((j >> 3) * 16, 16)] = p8
            acc = v
            if t % 2 == 0:
                acc = acc + u2 * plsc.load_gather(x_p, [col + (j >> 1)])
            if t % 4 == 0:
                acc = acc + u4 * plsc.load_gather(x_p, [col + (j >> 2)])
            if t % 8 == 0:
                acc = acc + u8 * plsc.load_gather(x_p, [col + (j >> 3)])
            if not (skip_first and t == 0):
                if 2 in downs:
                    acc = acc + d2 * p2_v[pl.ds(j * 16, 16)]
                if 4 in downs:
                    acc = acc + d4 * p4_v[pl.ds(j * 16, 16)]
                if 8 in downs:
                    acc = acc + d8 * p8_v[pl.ds(j * 16, 16)]
            plsc.store_scatter(o_p, [col + j], acc)

    def do_group(g, _):
        row0 = wid * rows_per_w + g * T
        pltpu.sync_copy(x_hbm.at[pl.ds(row0 * D, T * D)], x_v)

        @plsc.parallel_loop(0, D // 16, unroll=4)
        def respread(c):
            for r in range(T):
                v = x_v[pl.ds(r * D + c * 16, 16)]
                plsc.store_scatter(x_p, [r * P + c * 16 + lane], v)

        for lo, hi, downs in (
            (128, 256, ()),
            (64, 128, (2,)),
            (32, 64, (2, 4)),
            (16, 32, (2, 4, 8)),
            (8, 16, (2, 4, 8)),
            (4, 8, (2, 4, 8)),
            (2, 4, (2, 4, 8)),
        ):
            @plsc.parallel_loop(lo, hi, unroll=4)
            def sweep(jb, downs=downs):
                jblock(jb, downs)

        jblock(1, (2, 4, 8))
        jblock(0, (2, 4, 8), skip_first=True)

        @plsc.parallel_loop(0, D // 16, unroll=4)
        def unspread(c):
            for r in range(T):
                v = plsc.load_gather(o_p, [r * P + c * 16 + lane])
                x_v[pl.ds(r * D + c * 16, 16)] = v

        pltpu.sync_copy(x_v, o_hbm.at[pl.ds(row0 * D, T * D)])
        return 0

    lax.fori_loop(0, rows_per_w // T, do_group, 0)


def _tc_body(w_ref, x_ref, o_ref):
    xb = x_ref[...]
    xt = xb.T
    out = xt
    for i, s in enumerate(STRIDES):
        n = D // s
        uw = w_ref[i]
        dw = w_ref[3 + i]
        pref = (xt[:n] * uw)[:, None, :]
        dil = jnp.concatenate(
            [pref, jnp.zeros((n, s - 1, R), jnp.float32)], axis=1
        ).reshape(D, R)
        pooled = xt.reshape(n, s, R).sum(axis=1) * dw
        row = jax.lax.broadcasted_iota(jnp.int32, (n, R), 0)
        pooled = jnp.where(row >= 1, pooled, 0.0)
        down = jnp.concatenate(
            [pooled, jnp.zeros((D - n, R), jnp.float32)], axis=0
        )
        out = out + dil + down
    o_ref[...] = out.T


def _run_sc(wb, xflat, nrows):
    mesh = plsc.VectorSubcoreMesh(core_axis_name="c", subcore_axis_name="s")
    run = functools.partial(
        pl.kernel,
        mesh=mesh,
        compiler_params=pltpu.CompilerParams(needs_layout_passes=False),
        out_type=jax.ShapeDtypeStruct((nrows * D,), jnp.float32),
        scratch_types=[
            pltpu.VMEM((96,), jnp.float32),
            pltpu.VMEM((T * D,), jnp.float32),
            pltpu.VMEM((T * P,), jnp.float32),
            pltpu.VMEM((T * P,), jnp.float32),
            pltpu.VMEM((1024 * 16,), jnp.float32),
            pltpu.VMEM((512 * 16,), jnp.float32),
            pltpu.VMEM((256 * 16,), jnp.float32),
        ],
    )(functools.partial(_sc_body, nrows // 32))
    return run(wb, xflat)


def _run_tc(w, xf):
    nrows = xf.shape[0]
    return pl.pallas_call(
        _tc_body,
        grid=(nrows // R,),
        in_specs=[
            pl.BlockSpec(memory_space=pltpu.SMEM),
            pl.BlockSpec((R, D), lambda i: (i, 0)),
        ],
        out_specs=pl.BlockSpec((R, D), lambda i: (i, 0)),
        out_shape=jax.ShapeDtypeStruct((nrows, D), jnp.float32),
    )(w, xf)


def kernel(x, up_weights, down_weights):
    B, S, d = x.shape
    n = B * S
    xf = x.reshape(n, d)
    w = jnp.concatenate([jax.nn.sigmoid(up_weights), jax.nn.sigmoid(down_weights)])
    wb = jnp.broadcast_to(w[:, None], (6, 16)).reshape(96)
    out_sc = _run_sc(wb, xf[:SC_ROWS].reshape(SC_ROWS * d), SC_ROWS)
    out_tc = _run_tc(w, xf[SC_ROWS:])
    out = jnp.concatenate([out_sc.reshape(SC_ROWS, d), out_tc], axis=0)
    return out.reshape(B, S, d)


# speedup vs baseline: 4.1712x; 1.0009x over previous
import functools
import jax
import jax.numpy as jnp
from jax import lax
from jax.experimental import pallas as pl
from jax.experimental.pallas import tpu as pltpu
from jax.experimental.pallas import tpu_sc as plsc

D = 2048
P = 2049
T = 16
R = 256
STRIDES = (2, 4, 8)
SC_ROWS = 6656


def _sc_body(rows_per_w, w_hbm, x_hbm, o_hbm, w_v, x_v, x_p, o_p, p2_v, p4_v, p8_v):
    nc = 2
    wid = lax.axis_index("s") * nc + lax.axis_index("c")
    pltpu.sync_copy(w_hbm, w_v)
    u2 = w_v[pl.ds(0, 16)]
    u4 = w_v[pl.ds(16, 16)]
    u8 = w_v[pl.ds(32, 16)]
    d2 = w_v[pl.ds(48, 16)]
    d4 = w_v[pl.ds(64, 16)]
    d8 = w_v[pl.ds(80, 16)]
    lane = lax.iota(jnp.int32, 16)
    col = lane * P

    def jblock(jb, downs, skip_first=False):
        j0 = jb * 8
        p2 = p4 = p8 = None
        for t in (7, 6, 5, 4, 3, 2, 1, 0):
            j = j0 + t
            v = plsc.load_gather(x_p, [col + j])
            if t % 2 == 1:
                p2 = v
            else:
                p2 = p2 + v
                p2_v[pl.ds((j >> 1) * 16, 16)] = p2
                if t in (6, 2):
                    p4 = p2
                else:
                    p4 = p4 + p2
                    p4_v[pl.ds((j >> 2) * 16, 16)] = p4
                    if t == 4:
                        p8 = p4
                    else:
                        p8 = p8 + p4
                        p8_v[pl.ds((j >> 3) * 16, 16)] = p8
            acc = v
            if t % 2 == 0:
                acc = acc + u2 * plsc.load_gather(x_p, [col + (j >> 1)])
            if t % 4 == 0:
                acc = acc + u4 * plsc.load_gather(x_p, [col + (j >> 2)])
            if t % 8 == 0:
                acc = acc + u8 * plsc.load_gather(x_p, [col + (j >> 3)])
            if not (skip_first and t == 0):
                if 2 in downs:
                    acc = acc + d2 * p2_v[pl.ds(j * 16, 16)]
                if 4 in downs:
                    acc = acc + d4 * p4_v[pl.ds(j * 16, 16)]
                if 8 in downs:
                    acc = acc + d8 * p8_v[pl.ds(j * 16, 16)]
            plsc.store_scatter(o_p, [col + j], acc)

    def do_group(g, _):
        row0 = wid * rows_per_w + g * T
        pltpu.sync_copy(x_hbm.at[pl.ds(row0 * D, T * D)], x_v)

        @plsc.parallel_loop(0, D // 16, unroll=4)
        def respread(c):
            for r in range(T):
                v = x_v[pl.ds(r * D + c * 16, 16)]
                plsc.store_scatter(x_p, [r * P + c * 16 + lane], v)

        for lo, hi, downs in (
            (128, 256, ()),
            (64, 128, (2,)),
            (32, 64, (2, 4)),
            (16, 32, (2, 4, 8)),
            (8, 16, (2, 4, 8)),
            (4, 8, (2, 4, 8)),
            (2, 4, (2, 4, 8)),
        ):
            @plsc.parallel_loop(lo, hi, unroll=8)
            def sweep(jb, downs=downs):
                jblock(jb, downs)

        jblock(1, (2, 4, 8))
        jblock(0, (2, 4, 8), skip_first=True)

        @plsc.parallel_loop(0, D // 16, unroll=4)
        def unspread(c):
            for r in range(T):
                v = plsc.load_gather(o_p, [r * P + c * 16 + lane])
                x_v[pl.ds(r * D + c * 16, 16)] = v

        pltpu.sync_copy(x_v, o_hbm.at[pl.ds(row0 * D, T * D)])
        return 0

    lax.fori_loop(0, rows_per_w // T, do_group, 0)


def _tc_body(w_ref, x_ref, o_ref):
    xb = x_ref[...]
    xt = xb.T
    out = xt
    for i, s in enumerate(STRIDES):
        n = D // s
        uw = w_ref[i]
        dw = w_ref[3 + i]
        pref = (xt[:n] * uw)[:, None, :]
        dil = jnp.concatenate(
            [pref, jnp.zeros((n, s - 1, R), jnp.float32)], axis=1
        ).reshape(D, R)
        pooled = xt.reshape(n, s, R).sum(axis=1) * dw
        row = jax.lax.broadcasted_iota(jnp.int32, (n, R), 0)
        pooled = jnp.where(row >= 1, pooled, 0.0)
        down = jnp.concatenate(
            [pooled, jnp.zeros((D - n, R), jnp.float32)], axis=0
        )
        out = out + dil + down
    o_ref[...] = out.T


def _run_sc(wb, xflat, nrows):
    mesh = plsc.VectorSubcoreMesh(core_axis_name="c", subcore_axis_name="s")
    run = functools.partial(
        pl.kernel,
        mesh=mesh,
        compiler_params=pltpu.CompilerParams(needs_layout_passes=False),
        out_type=jax.ShapeDtypeStruct((nrows * D,), jnp.float32),
        scratch_types=[
            pltpu.VMEM((96,), jnp.float32),
            pltpu.VMEM((T * D,), jnp.float32),
            pltpu.VMEM((T * P,), jnp.float32),
            pltpu.VMEM((T * P,), jnp.float32),
            pltpu.VMEM((1024 * 16,), jnp.float32),
            pltpu.VMEM((512 * 16,), jnp.float32),
            pltpu.VMEM((256 * 16,), jnp.float32),
        ],
    )(functools.partial(_sc_body, nrows // 32))
    return run(wb, xflat)


def _run_tc(w, xf):
    nrows = xf.shape[0]
    return pl.pallas_call(
        _tc_body,
        grid=(nrows // R,),
        in_specs=[
            pl.BlockSpec(memory_space=pltpu.SMEM),
            pl.BlockSpec((R, D), lambda i: (i, 0)),
        ],
        out_specs=pl.BlockSpec((R, D), lambda i: (i, 0)),
        out_shape=jax.ShapeDtypeStruct((nrows, D), jnp.float32),
    )(w, xf)


def kernel(x, up_weights, down_weights):
    B, S, d = x.shape
    n = B * S
    xf = x.reshape(n, d)
    w = jnp.concatenate([jax.nn.sigmoid(up_weights), jax.nn.sigmoid(down_weights)])
    wb = jnp.broadcast_to(w[:, None], (6, 16)).reshape(96)
    out_sc = _run_sc(wb, xf[:SC_ROWS].reshape(SC_ROWS * d), SC_ROWS)
    out_tc = _run_tc(w, xf[SC_ROWS:])
    out = jnp.concatenate([out_sc.reshape(SC_ROWS, d), out_tc], axis=0)
    return out.reshape(B, S, d)
